# Initial kernel scaffold; baseline (speedup 1.0000x reference)
import jax
import jax.numpy as jnp
from jax.experimental import pallas as pl


def kernel(x, a, b, alpha, id_gain, bias):
    raise NotImplementedError("write your pallas kernel here")


# speedup vs baseline: 811.5986x; 811.5986x over previous
import jax
import jax.numpy as jnp
from jax import lax
from jax.experimental import pallas as pl
from jax.experimental.pallas import tpu as pltpu
from jax.experimental.pallas import tpu_sc as plsc

_C = 192
_K = 32
_HW = 224 * 224
_BC = 2 * _C
_N = _BC * _HW
_NW = 32
_IPW = _BC // _NW
_CHUNK = 12544
_NCHUNK = _HW // _CHUNK
_SEG = 48
_SCALE = (_K - 1) / 2.0


def _body(x_hbm, a_hbm, b_hbm, alpha_hbm, g_hbm, bias_hbm, out_hbm,
          in0, in1, ou0, ou1, alpha_v, a_v, b_v, g_v, bias_v,
          p0, p1, p2, p3, in_sem, out_sem):
    wid = lax.axis_index("s") * 2 + lax.axis_index("c")

    pltpu.sync_copy(alpha_hbm, alpha_v)
    pltpu.sync_copy(a_hbm, a_v)
    pltpu.sync_copy(b_hbm, b_v)
    pltpu.sync_copy(g_hbm, g_v)
    pltpu.sync_copy(bias_hbm, bias_v)

    iota = lax.iota(jnp.int32, 16)

    @pl.loop(0, _IPW)
    def _build(j):
        c = lax.rem(_IPW * wid + j, _C)
        c_splat = jnp.full((16,), c, dtype=jnp.int32)
        bias_s = plsc.load_gather(bias_v, [c_splat])
        for k in range(_SEG // 16):
            s = iota + (16 * k)
            i = s - 2
            i0 = jnp.maximum(jnp.minimum(i - 1, _K - 1), 0)
            i1 = jnp.maximum(jnp.minimum(i, _K - 1), 0)
            i2 = jnp.maximum(jnp.minimum(i + 1, _K - 1), 0)
            i3 = jnp.maximum(jnp.minimum(i + 2, _K - 1), 0)
            a0 = plsc.load_gather(alpha_v, [c_splat, i0])
            a1 = plsc.load_gather(alpha_v, [c_splat, i1])
            a2 = plsc.load_gather(alpha_v, [c_splat, i2])
            a3 = plsc.load_gather(alpha_v, [c_splat, i3])
            sl = pl.ds(j * _SEG + 16 * k, 16)
            p0[sl] = (a0 + 4.0 * a1 + a2) * (1.0 / 6.0) + bias_s
            p1[sl] = (a2 - a0) * 0.5
            p2[sl] = (a0 - 2.0 * a1 + a2) * 0.5
            p3[sl] = (-a0 + 3.0 * a1 - 3.0 * a2 + a3) * (1.0 / 6.0)

    nslots = _IPW * _NCHUNK
    base0 = _IPW * wid * _HW

    def chunk_base(slot):
        return base0 + slot * _CHUNK

    bufs = ((in0, ou0), (in1, ou1))

    pltpu.async_copy(x_hbm.at[pl.ds(chunk_base(0), _CHUNK)], in0,
                     in_sem.at[0])
    pltpu.async_copy(x_hbm.at[pl.ds(chunk_base(1), _CHUNK)], in1,
                     in_sem.at[1])

    @pl.loop(0, nslots)
    def _main(slot):
        img = slot // _NCHUNK
        c = lax.rem(_IPW * wid + img, _C)
        c_splat = jnp.full((16,), c, dtype=jnp.int32)
        A_s = plsc.load_gather(a_v, [c_splat]) * _SCALE
        B_s = plsc.load_gather(b_v, [c_splat]) * _SCALE + (_SCALE + 2.0)
        G_s = plsc.load_gather(g_v, [c_splat])
        sbase = img * _SEG
        base = chunk_base(slot)

        def run(bi):
            inb, oub = bufs[bi]
            pltpu.make_async_copy(x_hbm.at[pl.ds(0, _CHUNK)], inb,
                                  in_sem.at[bi]).wait()
            @pl.when(slot >= 2)
            def _():
                pltpu.make_async_copy(oub, out_hbm.at[pl.ds(0, _CHUNK)],
                                      out_sem.at[bi]).wait()

            @pl.loop(0, _CHUNK // 16, unroll=8)
            def _inner(it):
                off = it * 16
                xv = inb[pl.ds(off, 16)]
                u = xv * A_s + B_s
                u = jnp.minimum(jnp.maximum(u, 0.0), 35.0)
                si = u.astype(jnp.int32)
                t = u - si.astype(jnp.float32)
                idx = si + sbase
                q3 = plsc.load_gather(p3, [idx])
                q2 = plsc.load_gather(p2, [idx])
                q1 = plsc.load_gather(p1, [idx])
                q0 = plsc.load_gather(p0, [idx])
                r = ((q3 * t + q2) * t + q1) * t + q0
                oub[pl.ds(off, 16)] = xv * G_s + r

            pltpu.async_copy(oub, out_hbm.at[pl.ds(base, _CHUNK)],
                             out_sem.at[bi])

            @pl.when(slot + 2 < nslots)
            def _():
                pltpu.async_copy(
                    x_hbm.at[pl.ds(chunk_base(slot + 2), _CHUNK)], inb,
                    in_sem.at[bi])

        @pl.when(lax.rem(slot, 2) == 0)
        def _():
            run(0)

        @pl.when(lax.rem(slot, 2) != 0)
        def _():
            run(1)

    pltpu.make_async_copy(ou0, out_hbm.at[pl.ds(0, _CHUNK)],
                          out_sem.at[0]).wait()
    pltpu.make_async_copy(ou1, out_hbm.at[pl.ds(0, _CHUNK)],
                          out_sem.at[1]).wait()


_kernel_call = pl.kernel(
    _body,
    out_type=jax.ShapeDtypeStruct((_N,), jnp.float32),
    mesh=plsc.VectorSubcoreMesh(core_axis_name="c", subcore_axis_name="s"),
    compiler_params=pltpu.CompilerParams(needs_layout_passes=False),
    scratch_types=[
        pltpu.VMEM((_CHUNK,), jnp.float32),
        pltpu.VMEM((_CHUNK,), jnp.float32),
        pltpu.VMEM((_CHUNK,), jnp.float32),
        pltpu.VMEM((_CHUNK,), jnp.float32),
        pltpu.VMEM((_C, _K), jnp.float32),
        pltpu.VMEM((_C,), jnp.float32),
        pltpu.VMEM((_C,), jnp.float32),
        pltpu.VMEM((_C,), jnp.float32),
        pltpu.VMEM((_C,), jnp.float32),
        pltpu.VMEM((_IPW * _SEG,), jnp.float32),
        pltpu.VMEM((_IPW * _SEG,), jnp.float32),
        pltpu.VMEM((_IPW * _SEG,), jnp.float32),
        pltpu.VMEM((_IPW * _SEG,), jnp.float32),
        pltpu.SemaphoreType.DMA((2,)),
        pltpu.SemaphoreType.DMA((2,)),
    ],
)


@jax.jit
def kernel(x, a, b, alpha, id_gain, bias):
    y = _kernel_call(x.reshape(-1), a, b, alpha, id_gain, bias)
    return y.reshape(x.shape)


# speedup vs baseline: 3704.4664x; 4.5644x over previous
import jax
import jax.numpy as jnp
from jax import lax
from jax.experimental import pallas as pl
from jax.experimental.pallas import tpu as pltpu
from jax.experimental.pallas import tpu_sc as plsc

_C = 192
_K = 32
_H = 224
_W = 224
_BC = 2 * _C
_NW = 32
_IPW = _BC // _NW
_ROWS = 56
_NCHUNK = _H // _ROWS
_CVEC = _W // 16
_SEG = 48
_SCALE = (_K - 1) / 2.0


def _body(x_hbm, a_hbm, b_hbm, alpha_hbm, g_hbm, bias_hbm, out_hbm,
          in0, in1, ou0, ou1, alpha_v, a_v, b_v, g_v, bias_v,
          p0, p1, p2, p3, in_sem, out_sem):
    wid = lax.axis_index("s") * 2 + lax.axis_index("c")

    pltpu.sync_copy(alpha_hbm, alpha_v)
    pltpu.sync_copy(a_hbm, a_v)
    pltpu.sync_copy(b_hbm, b_v)
    pltpu.sync_copy(g_hbm, g_v)
    pltpu.sync_copy(bias_hbm, bias_v)

    iota = lax.iota(jnp.int32, 16)

    @pl.loop(0, _IPW)
    def _build(j):
        c = lax.rem(_IPW * wid + j, _C)
        c_splat = jnp.full((16,), c, dtype=jnp.int32)
        bias_s = plsc.load_gather(bias_v, [c_splat])
        for k in range(_SEG // 16):
            s = iota + (16 * k)
            i = s - 2
            i0 = jnp.maximum(jnp.minimum(i - 1, _K - 1), 0)
            i1 = jnp.maximum(jnp.minimum(i, _K - 1), 0)
            i2 = jnp.maximum(jnp.minimum(i + 1, _K - 1), 0)
            i3 = jnp.maximum(jnp.minimum(i + 2, _K - 1), 0)
            a0 = plsc.load_gather(alpha_v, [c_splat, i0])
            a1 = plsc.load_gather(alpha_v, [c_splat, i1])
            a2 = plsc.load_gather(alpha_v, [c_splat, i2])
            a3 = plsc.load_gather(alpha_v, [c_splat, i3])
            sl = pl.ds(j * _SEG + 16 * k, 16)
            p0[sl] = (a0 + 4.0 * a1 + a2) * (1.0 / 6.0) + bias_s
            p1[sl] = (a2 - a0) * 0.5
            p2[sl] = (a0 - 2.0 * a1 + a2) * 0.5
            p3[sl] = (-a0 + 3.0 * a1 - 3.0 * a2 + a3) * (1.0 / 6.0)

    nslots = _IPW * _NCHUNK

    def chunk_coords(slot):
        img = slot // _NCHUNK
        h0 = lax.rem(slot, _NCHUNK) * _ROWS
        bc = _IPW * wid + img
        return bc // _C, lax.rem(bc, _C), h0

    bufs = ((in0, ou0), (in1, ou1))

    def start_in(slot, bi):
        bb, cc, h0 = chunk_coords(slot)
        pltpu.async_copy(x_hbm.at[bb, cc, pl.ds(h0, _ROWS)], bufs[bi][0],
                         in_sem.at[bi])

    start_in(0, 0)
    start_in(1, 1)

    @pl.loop(0, nslots)
    def _main(slot):
        img = slot // _NCHUNK
        c = lax.rem(_IPW * wid + img, _C)
        c_splat = jnp.full((16,), c, dtype=jnp.int32)
        A_s = plsc.load_gather(a_v, [c_splat]) * _SCALE
        B_s = plsc.load_gather(b_v, [c_splat]) * _SCALE + (_SCALE + 2.0)
        G_s = plsc.load_gather(g_v, [c_splat])
        sbase = img * _SEG
        bb, cc, h0 = chunk_coords(slot)

        def run(bi):
            inb, oub = bufs[bi]
            pltpu.make_async_copy(x_hbm.at[0, 0, pl.ds(0, _ROWS)], inb,
                                  in_sem.at[bi]).wait()
            @pl.when(slot >= 2)
            def _():
                pltpu.make_async_copy(oub, out_hbm.at[0, 0, pl.ds(0, _ROWS)],
                                      out_sem.at[bi]).wait()

            @plsc.parallel_loop(0, _ROWS * _CVEC, unroll=8)
            def _inner(it):
                r = it // _CVEC
                c0 = lax.rem(it, _CVEC) * 16
                xv = inb[r, pl.ds(c0, 16)]
                u = xv * A_s + B_s
                u = jnp.minimum(jnp.maximum(u, 0.0), 35.0)
                si = u.astype(jnp.int32)
                t = u - si.astype(jnp.float32)
                idx = si + sbase
                q3 = plsc.load_gather(p3, [idx])
                q2 = plsc.load_gather(p2, [idx])
                q1 = plsc.load_gather(p1, [idx])
                q0 = plsc.load_gather(p0, [idx])
                r_ = ((q3 * t + q2) * t + q1) * t + q0
                oub[r, pl.ds(c0, 16)] = xv * G_s + r_

            pltpu.async_copy(oub, out_hbm.at[bb, cc, pl.ds(h0, _ROWS)],
                             out_sem.at[bi])

            @pl.when(slot + 2 < nslots)
            def _():
                start_in(slot + 2, bi)

        @pl.when(lax.rem(slot, 2) == 0)
        def _():
            run(0)

        @pl.when(lax.rem(slot, 2) != 0)
        def _():
            run(1)

    pltpu.make_async_copy(ou0, out_hbm.at[0, 0, pl.ds(0, _ROWS)],
                          out_sem.at[0]).wait()
    pltpu.make_async_copy(ou1, out_hbm.at[0, 0, pl.ds(0, _ROWS)],
                          out_sem.at[1]).wait()


_kernel_call = pl.kernel(
    _body,
    out_type=jax.ShapeDtypeStruct((2, _C, _H, _W), jnp.float32),
    mesh=plsc.VectorSubcoreMesh(core_axis_name="c", subcore_axis_name="s"),
    compiler_params=pltpu.CompilerParams(needs_layout_passes=False),
    scratch_types=[
        pltpu.VMEM((_ROWS, _W), jnp.float32),
        pltpu.VMEM((_ROWS, _W), jnp.float32),
        pltpu.VMEM((_ROWS, _W), jnp.float32),
        pltpu.VMEM((_ROWS, _W), jnp.float32),
        pltpu.VMEM((_C, _K), jnp.float32),
        pltpu.VMEM((_C,), jnp.float32),
        pltpu.VMEM((_C,), jnp.float32),
        pltpu.VMEM((_C,), jnp.float32),
        pltpu.VMEM((_C,), jnp.float32),
        pltpu.VMEM((_IPW * _SEG,), jnp.float32),
        pltpu.VMEM((_IPW * _SEG,), jnp.float32),
        pltpu.VMEM((_IPW * _SEG,), jnp.float32),
        pltpu.VMEM((_IPW * _SEG,), jnp.float32),
        pltpu.SemaphoreType.DMA((2,)),
        pltpu.SemaphoreType.DMA((2,)),
    ],
)


@jax.jit
def kernel(x, a, b, alpha, id_gain, bias):
    return _kernel_call(x, a, b, alpha, id_gain, bias)


# speedup vs baseline: 4040.9294x; 1.0908x over previous
import jax
import jax.numpy as jnp
from jax import lax
from jax.experimental import pallas as pl
from jax.experimental.pallas import tpu as pltpu
from jax.experimental.pallas import tpu_sc as plsc

_C = 192
_K = 32
_H = 224
_W = 224
_BC = 2 * _C
_NW = 32
_IPW = _BC // _NW
_ROWS = 56
_NCHUNK = _H // _ROWS
_CVEC = _W // 16
_SEG = 48
_SCALE = (_K - 1) / 2.0


def _body(x_hbm, a_hbm, b_hbm, alpha_hbm, g_hbm, bias_hbm, out_hbm,
          in0, in1, ou0, ou1, alpha_v, a_v, b_v, g_v, bias_v,
          p0, p1, p2, p3, in_sem, out_sem):
    wid = lax.axis_index("s") * 2 + lax.axis_index("c")

    pltpu.sync_copy(alpha_hbm, alpha_v)
    pltpu.sync_copy(a_hbm, a_v)
    pltpu.sync_copy(b_hbm, b_v)
    pltpu.sync_copy(g_hbm, g_v)
    pltpu.sync_copy(bias_hbm, bias_v)

    iota = lax.iota(jnp.int32, 16)

    @pl.loop(0, _IPW)
    def _build(j):
        c = lax.rem(_IPW * wid + j, _C)
        c_splat = jnp.full((16,), c, dtype=jnp.int32)
        bias_s = plsc.load_gather(bias_v, [c_splat])
        for k in range(_SEG // 16):
            s = iota + (16 * k)
            i = s - 2
            i0 = jnp.maximum(jnp.minimum(i - 1, _K - 1), 0)
            i1 = jnp.maximum(jnp.minimum(i, _K - 1), 0)
            i2 = jnp.maximum(jnp.minimum(i + 1, _K - 1), 0)
            i3 = jnp.maximum(jnp.minimum(i + 2, _K - 1), 0)
            a0 = plsc.load_gather(alpha_v, [c_splat, i0])
            a1 = plsc.load_gather(alpha_v, [c_splat, i1])
            a2 = plsc.load_gather(alpha_v, [c_splat, i2])
            a3 = plsc.load_gather(alpha_v, [c_splat, i3])
            sl = pl.ds(j * _SEG + 16 * k, 16)
            p0[sl] = (a0 + 4.0 * a1 + a2) * (1.0 / 6.0) + bias_s
            p1[sl] = (a2 - a0) * 0.5
            p2[sl] = (a0 - 2.0 * a1 + a2) * 0.5
            p3[sl] = (-a0 + 3.0 * a1 - 3.0 * a2 + a3) * (1.0 / 6.0)

    nslots = _IPW * _NCHUNK

    def chunk_coords(slot):
        img = slot // _NCHUNK
        h0 = lax.rem(slot, _NCHUNK) * _ROWS
        bc = _IPW * wid + img
        return bc // _C, lax.rem(bc, _C), h0

    bufs = ((in0, ou0), (in1, ou1))

    def start_in(slot, bi):
        bb, cc, h0 = chunk_coords(slot)
        pltpu.async_copy(x_hbm.at[bb, cc, pl.ds(h0, _ROWS)], bufs[bi][0],
                         in_sem.at[bi])

    start_in(0, 0)
    start_in(1, 1)

    @pl.loop(0, nslots)
    def _main(slot):
        img = slot // _NCHUNK
        c = lax.rem(_IPW * wid + img, _C)
        c_splat = jnp.full((16,), c, dtype=jnp.int32)
        A_s = plsc.load_gather(a_v, [c_splat]) * _SCALE
        B_s = plsc.load_gather(b_v, [c_splat]) * _SCALE + (_SCALE + 2.0)
        G_s = plsc.load_gather(g_v, [c_splat])
        sbase = img * _SEG
        bb, cc, h0 = chunk_coords(slot)

        def run(bi):
            inb, oub = bufs[bi]
            pltpu.make_async_copy(x_hbm.at[0, 0, pl.ds(0, _ROWS)], inb,
                                  in_sem.at[bi]).wait()
            @pl.when(slot >= 2)
            def _():
                pltpu.make_async_copy(oub, out_hbm.at[0, 0, pl.ds(0, _ROWS)],
                                      out_sem.at[bi]).wait()

            p0s = p0.at[pl.ds(sbase, _SEG)]
            p1s = p1.at[pl.ds(sbase, _SEG)]
            p2s = p2.at[pl.ds(sbase, _SEG)]
            p3s = p3.at[pl.ds(sbase, _SEG)]

            @plsc.parallel_loop(0, 2 * _ROWS, unroll=2)
            def _inner(it):
                r = it >> 1
                cb = (it & 1) * 7
                for cj in range(_CVEC // 2):
                    ci = cb + cj
                    xv = inb[r, pl.ds(16 * ci, 16)]
                    u = xv * A_s + B_s
                    u = jnp.minimum(jnp.maximum(u, 0.0), 35.0)
                    idx = u.astype(jnp.int32)
                    t = u - idx.astype(jnp.float32)
                    q3 = plsc.load_gather(p3s, [idx])
                    q2 = plsc.load_gather(p2s, [idx])
                    q1 = plsc.load_gather(p1s, [idx])
                    q0 = plsc.load_gather(p0s, [idx])
                    r_ = ((q3 * t + q2) * t + q1) * t + q0
                    oub[r, pl.ds(16 * ci, 16)] = xv * G_s + r_

            pltpu.async_copy(oub, out_hbm.at[bb, cc, pl.ds(h0, _ROWS)],
                             out_sem.at[bi])

            @pl.when(slot + 2 < nslots)
            def _():
                start_in(slot + 2, bi)

        @pl.when(lax.rem(slot, 2) == 0)
        def _():
            run(0)

        @pl.when(lax.rem(slot, 2) != 0)
        def _():
            run(1)

    pltpu.make_async_copy(ou0, out_hbm.at[0, 0, pl.ds(0, _ROWS)],
                          out_sem.at[0]).wait()
    pltpu.make_async_copy(ou1, out_hbm.at[0, 0, pl.ds(0, _ROWS)],
                          out_sem.at[1]).wait()


_kernel_call = pl.kernel(
    _body,
    out_type=jax.ShapeDtypeStruct((2, _C, _H, _W), jnp.float32),
    mesh=plsc.VectorSubcoreMesh(core_axis_name="c", subcore_axis_name="s"),
    compiler_params=pltpu.CompilerParams(needs_layout_passes=False),
    scratch_types=[
        pltpu.VMEM((_ROWS, _W), jnp.float32),
        pltpu.VMEM((_ROWS, _W), jnp.float32),
        pltpu.VMEM((_ROWS, _W), jnp.float32),
        pltpu.VMEM((_ROWS, _W), jnp.float32),
        pltpu.VMEM((_C, _K), jnp.float32),
        pltpu.VMEM((_C,), jnp.float32),
        pltpu.VMEM((_C,), jnp.float32),
        pltpu.VMEM((_C,), jnp.float32),
        pltpu.VMEM((_C,), jnp.float32),
        pltpu.VMEM((_IPW * _SEG,), jnp.float32),
        pltpu.VMEM((_IPW * _SEG,), jnp.float32),
        pltpu.VMEM((_IPW * _SEG,), jnp.float32),
        pltpu.VMEM((_IPW * _SEG,), jnp.float32),
        pltpu.SemaphoreType.DMA((2,)),
        pltpu.SemaphoreType.DMA((2,)),
    ],
)


@jax.jit
def kernel(x, a, b, alpha, id_gain, bias):
    return _kernel_call(x, a, b, alpha, id_gain, bias)
